# Initial kernel scaffold; baseline (speedup 1.0000x reference)
import jax
import jax.numpy as jnp
from jax.experimental import pallas as pl


def kernel(x, edge_index, batch, y_pred, W_embed, b_embed, W1_0, b1_0, W2_0, b2_0, W1_1, b1_1, W2_1, b2_1, Wf1, bf1, gamma, beta, Wf2, bf2):
    raise NotImplementedError("write your pallas kernel here")


# speedup vs baseline: 4.7571x; 4.7571x over previous
import functools

import jax
import jax.numpy as jnp
from jax import lax
from jax.experimental import pallas as pl
from jax.experimental.pallas import tpu as pltpu
from jax.experimental.pallas import tpu_sc as plsc

_NC = 2
_NS = 16
_ROWS = 400


def _make_seg_sum(n, e, h):
  nw = _NC * _NS
  epw = e // nw
  chunk = 80
  steps = epw // chunk
  assert epw % chunk == 0
  spl = -(-n // _NS) // 8 * 8 + 8
  last = n - (_NS - 1) * spl
  assert last > 0 and last % 8 == 0

  def body(h_hbm, src_hbm, dst_hbm, zeros_hbm, out_hbm,
           acc, src_v, dst_v, rows_v, sem):
    cid = lax.axis_index("c")
    sid = lax.axis_index("s")
    row0 = pl.multiple_of(sid * spl, 8)

    def seed(rows):
      @pl.when(cid == 0)
      def _():
        pltpu.sync_copy(h_hbm.at[pl.ds(row0, rows)],
                        acc.at[pl.ds(row0, rows)])

      @pl.when(cid != 0)
      def _():
        pltpu.sync_copy(zeros_hbm.at[pl.ds(row0, rows)],
                        acc.at[pl.ds(row0, rows)])

    @pl.when(sid < _NS - 1)
    def _():
      seed(spl)

    @pl.when(sid == _NS - 1)
    def _():
      seed(last)

    plsc.subcore_barrier()

    wid = sid * _NC + cid
    base = wid * epw

    def step(i, carry):
      off = pl.multiple_of(base + i * chunk, 8)
      pltpu.sync_copy(src_hbm.at[pl.ds(off, chunk)], src_v)
      pltpu.sync_copy(dst_hbm.at[pl.ds(off, chunk)], dst_v)
      pltpu.async_copy(h_hbm.at[src_v], rows_v, sem).wait()
      pltpu.sync_copy(rows_v, acc.at[dst_v], add=True)
      return carry

    lax.fori_loop(0, steps, step, 0)
    plsc.subcore_barrier()

    def writeout(rows):
      pltpu.sync_copy(acc.at[pl.ds(row0, rows)],
                      out_hbm.at[cid, pl.ds(row0, rows)])

    @pl.when(sid < _NS - 1)
    def _():
      writeout(spl)

    @pl.when(sid == _NS - 1)
    def _():
      writeout(last)

  return pl.kernel(
      body,
      out_type=jax.ShapeDtypeStruct((_NC, n, h), jnp.float32),
      mesh=plsc.VectorSubcoreMesh(core_axis_name="c", subcore_axis_name="s",
                                  num_cores=_NC, num_subcores=_NS),
      scratch_types=[
          pltpu.VMEM_SHARED((n, h), jnp.float32),
          pltpu.VMEM((chunk,), jnp.int32),
          pltpu.VMEM((chunk,), jnp.int32),
          pltpu.VMEM((chunk, h), jnp.float32),
          pltpu.SemaphoreType.DMA,
      ],
  )


def _dotf(a, b):
  return jnp.dot(a, b, preferred_element_type=jnp.float32)


def _embed_body(x_ref, w_ref, b_ref, o_ref):
  o_ref[...] = _dotf(x_ref[...], w_ref[...]) + b_ref[...]


def _gin_body(agg_ref, w1_ref, b1_ref, w2_ref, b2_ref, o_ref):
  t = agg_ref[0] + agg_ref[1]
  t = jnp.maximum(_dotf(t, w1_ref[...]) + b1_ref[...], 0.0)
  o_ref[...] = _dotf(t, w2_ref[...]) + b2_ref[...]


def _gin_fuse_body(n_tiles, g, agg_ref, w1_ref, b1_ref, w2_ref, b2_ref,
                   batch_ref, ypred_ref, wf1a_ref, wf1b_ref, bf1_ref,
                   z_ref, sums_ref):
  i = pl.program_id(0)
  t = agg_ref[0] + agg_ref[1]
  t = jnp.maximum(_dotf(t, w1_ref[...]) + b1_ref[...], 0.0)
  h2 = _dotf(t, w2_ref[...]) + b2_ref[...]
  yproj = _dotf(ypred_ref[...], wf1b_ref[...])
  b = batch_ref[0, 0, :]
  onehot = (b[:, None] ==
            lax.broadcasted_iota(jnp.int32, (b.shape[0], g), 1)
            ).astype(jnp.float32)
  z = _dotf(h2, wf1a_ref[...]) + _dotf(onehot, yproj) + bf1_ref[...]
  z_ref[...] = z

  @pl.when(i == 0)
  def _():
    sums_ref[...] = jnp.zeros_like(sums_ref)

  s1 = jnp.sum(z, axis=0)
  s2 = jnp.sum(z * z, axis=0)
  sums_ref[...] = sums_ref[...] + jnp.concatenate(
      [s1[None, :], s2[None, :]], axis=0)


def _fin_body(n, z_ref, sums_ref, gamma_ref, beta_ref, wf2_ref, bf2_ref,
              o_ref):
  mean = sums_ref[0, :] / n
  var = sums_ref[1, :] / n - mean * mean
  scale = lax.rsqrt(var + 1e-5) * gamma_ref[...]
  zn = (z_ref[...] - mean) * scale + beta_ref[...]
  zn = jnp.maximum(zn, 0.0)
  o = _dotf(zn, wf2_ref[...]) + bf2_ref[...]
  o_ref[...] = jax.nn.sigmoid(o)


def _full(shape):
  nd = len(shape)
  return pl.BlockSpec(shape, lambda i: (0,) * nd)


def kernel(x, edge_index, batch, y_pred, W_embed, b_embed,
           W1_0, b1_0, W2_0, b2_0, W1_1, b1_1, W2_1, b2_1,
           Wf1, bf1, gamma, beta, Wf2, bf2):
  n, d = x.shape
  h = W_embed.shape[1]
  e = edge_index.shape[1]
  g, out_dim = y_pred.shape
  h2w = 2 * h
  rows = _ROWS
  n_tiles = n // rows
  assert n % rows == 0

  src = edge_index[0]
  dst = edge_index[1]
  zeros = jnp.zeros((n, h), jnp.float32)
  batch3 = batch.reshape(n_tiles, 1, rows)
  wf1a = Wf1[:h]
  wf1b = Wf1[h:]

  seg_sum = _make_seg_sum(n, e, h)

  row_spec = pl.BlockSpec((rows, h), lambda i: (i, 0))
  agg_spec = pl.BlockSpec((_NC, rows, h), lambda i: (0, i, 0))

  h0 = pl.pallas_call(
      _embed_body,
      grid=(n_tiles,),
      in_specs=[pl.BlockSpec((rows, d), lambda i: (i, 0)),
                _full((d, h)), _full((h,))],
      out_specs=row_spec,
      out_shape=jax.ShapeDtypeStruct((n, h), jnp.float32),
  )(x, W_embed, b_embed)

  agg0 = seg_sum(h0, src, dst, zeros)

  h1 = pl.pallas_call(
      _gin_body,
      grid=(n_tiles,),
      in_specs=[agg_spec, _full((h, h2w)), _full((h2w,)),
                _full((h2w, h)), _full((h,))],
      out_specs=row_spec,
      out_shape=jax.ShapeDtypeStruct((n, h), jnp.float32),
  )(agg0, W1_0, b1_0, W2_0, b2_0)

  agg1 = seg_sum(h1, src, dst, zeros)

  z, sums = pl.pallas_call(
      functools.partial(_gin_fuse_body, n_tiles, g),
      grid=(n_tiles,),
      in_specs=[agg_spec, _full((h, h2w)), _full((h2w,)),
                _full((h2w, h)), _full((h,)),
                pl.BlockSpec((1, 1, rows), lambda i: (i, 0, 0)),
                _full((g, out_dim)), _full((h, h2w)),
                _full((out_dim, h2w)), _full((h2w,))],
      out_specs=[pl.BlockSpec((rows, h2w), lambda i: (i, 0)),
                 _full((2, h2w))],
      out_shape=[jax.ShapeDtypeStruct((n, h2w), jnp.float32),
                 jax.ShapeDtypeStruct((2, h2w), jnp.float32)],
  )(agg1, W1_1, b1_1, W2_1, b2_1, batch3, y_pred, wf1a, wf1b, bf1)

  node_score = pl.pallas_call(
      functools.partial(_fin_body, float(n)),
      grid=(n_tiles,),
      in_specs=[pl.BlockSpec((rows, h2w), lambda i: (i, 0)),
                _full((2, h2w)), _full((h2w,)), _full((h2w,)),
                _full((h2w, h)), _full((h,))],
      out_specs=row_spec,
      out_shape=jax.ShapeDtypeStruct((n, h), jnp.float32),
  )(z, sums, gamma, beta, Wf2, bf2)

  return node_score


# speedup vs baseline: 9.3233x; 1.9599x over previous
import functools

import jax
import jax.numpy as jnp
from jax import lax
from jax.experimental import pallas as pl
from jax.experimental.pallas import tpu as pltpu
from jax.experimental.pallas import tpu_sc as plsc

_NC = 2
_NS = 16
_ROWS = 400


def _make_seg_sum(n, e, h):
  nw = _NC * _NS
  epw = e // nw
  chunk = 80
  steps = epw // chunk
  sbs = 25
  assert epw % chunk == 0 and steps % sbs == 0
  spl = -(-n // _NS) // 8 * 8 + 8
  last = n - (_NS - 1) * spl
  assert last > 0 and last % 8 == 0

  def body(h_hbm, src_hbm, dst_hbm, zeros_hbm, out_hbm,
           acc, src_v, dst_v, rows_a, rows_b, sem_a, sem_b):
    cid = lax.axis_index("c")
    sid = lax.axis_index("s")
    row0 = pl.multiple_of(sid * spl, 8)

    def seed(rows):
      @pl.when(cid == 0)
      def _():
        pltpu.sync_copy(h_hbm.at[pl.ds(row0, rows)],
                        acc.at[pl.ds(row0, rows)])

      @pl.when(cid != 0)
      def _():
        pltpu.sync_copy(zeros_hbm.at[pl.ds(row0, rows)],
                        acc.at[pl.ds(row0, rows)])

    @pl.when(sid < _NS - 1)
    def _():
      seed(spl)

    @pl.when(sid == _NS - 1)
    def _():
      seed(last)

    plsc.subcore_barrier()

    wid = sid * _NC + cid

    def start(i, buf, sem):
      pltpu.async_copy(h_hbm.at[src_v.at[i]], buf, sem)

    def finish(i, buf, sem):
      pltpu.make_async_copy(h_hbm.at[src_v.at[i]], buf, sem).wait()
      pltpu.sync_copy(buf, acc.at[dst_v.at[i]], add=True)

    def superblock(sb, carry):
      pltpu.sync_copy(src_hbm.at[wid, sb], src_v)
      pltpu.sync_copy(dst_hbm.at[wid, sb], dst_v)
      start(0, rows_a, sem_a)

      def pair(j, c2):
        i0 = j * 2
        i1 = i0 + 1

        @pl.when(i1 < sbs)
        def _():
          start(i1, rows_b, sem_b)

        finish(i0, rows_a, sem_a)

        @pl.when(i1 < sbs)
        def _():
          @pl.when(i1 + 1 < sbs)
          def _():
            start(i1 + 1, rows_a, sem_a)

          finish(i1, rows_b, sem_b)

        return c2

      lax.fori_loop(0, (sbs + 1) // 2, pair, 0)
      return carry

    lax.fori_loop(0, steps // sbs, superblock, 0)
    plsc.subcore_barrier()

    def writeout(rows):
      pltpu.sync_copy(acc.at[pl.ds(row0, rows)],
                      out_hbm.at[cid, pl.ds(row0, rows)])

    @pl.when(sid < _NS - 1)
    def _():
      writeout(spl)

    @pl.when(sid == _NS - 1)
    def _():
      writeout(last)

  return pl.kernel(
      body,
      out_type=jax.ShapeDtypeStruct((_NC, n, h), jnp.float32),
      mesh=plsc.VectorSubcoreMesh(core_axis_name="c", subcore_axis_name="s",
                                  num_cores=_NC, num_subcores=_NS),
      scratch_types=[
          pltpu.VMEM_SHARED((n, h), jnp.float32),
          pltpu.VMEM((sbs, chunk), jnp.int32),
          pltpu.VMEM((sbs, chunk), jnp.int32),
          pltpu.VMEM((chunk, h), jnp.float32),
          pltpu.VMEM((chunk, h), jnp.float32),
          pltpu.SemaphoreType.DMA,
          pltpu.SemaphoreType.DMA,
      ],
  )


def _dotf(a, b):
  return jnp.dot(a, b, preferred_element_type=jnp.float32)


def _embed_body(x_ref, w_ref, b_ref, o_ref):
  o_ref[...] = _dotf(x_ref[...], w_ref[...]) + b_ref[...]


def _gin_body(agg_ref, w1_ref, b1_ref, w2_ref, b2_ref, o_ref):
  t = agg_ref[0] + agg_ref[1]
  t = jnp.maximum(_dotf(t, w1_ref[...]) + b1_ref[...], 0.0)
  o_ref[...] = _dotf(t, w2_ref[...]) + b2_ref[...]


def _gin_fuse_body(n_tiles, g, agg_ref, w1_ref, b1_ref, w2_ref, b2_ref,
                   batch_ref, ypred_ref, wf1a_ref, wf1b_ref, bf1_ref,
                   z_ref, sums_ref):
  i = pl.program_id(0)
  t = agg_ref[0] + agg_ref[1]
  t = jnp.maximum(_dotf(t, w1_ref[...]) + b1_ref[...], 0.0)
  h2 = _dotf(t, w2_ref[...]) + b2_ref[...]
  yproj = _dotf(ypred_ref[...], wf1b_ref[...])
  b = batch_ref[0, 0, :]
  onehot = (b[:, None] ==
            lax.broadcasted_iota(jnp.int32, (b.shape[0], g), 1)
            ).astype(jnp.float32)
  z = _dotf(h2, wf1a_ref[...]) + _dotf(onehot, yproj) + bf1_ref[...]
  z_ref[...] = z

  @pl.when(i == 0)
  def _():
    sums_ref[...] = jnp.zeros_like(sums_ref)

  s1 = jnp.sum(z, axis=0)
  s2 = jnp.sum(z * z, axis=0)
  sums_ref[...] = sums_ref[...] + jnp.concatenate(
      [s1[None, :], s2[None, :]], axis=0)


def _fin_body(n, z_ref, sums_ref, gamma_ref, beta_ref, wf2_ref, bf2_ref,
              o_ref):
  mean = sums_ref[0, :] / n
  var = sums_ref[1, :] / n - mean * mean
  scale = lax.rsqrt(var + 1e-5) * gamma_ref[...]
  zn = (z_ref[...] - mean) * scale + beta_ref[...]
  zn = jnp.maximum(zn, 0.0)
  o = _dotf(zn, wf2_ref[...]) + bf2_ref[...]
  o_ref[...] = jax.nn.sigmoid(o)


def _full(shape):
  nd = len(shape)
  return pl.BlockSpec(shape, lambda i: (0,) * nd)


def kernel(x, edge_index, batch, y_pred, W_embed, b_embed,
           W1_0, b1_0, W2_0, b2_0, W1_1, b1_1, W2_1, b2_1,
           Wf1, bf1, gamma, beta, Wf2, bf2):
  n, d = x.shape
  h = W_embed.shape[1]
  e = edge_index.shape[1]
  g, out_dim = y_pred.shape
  h2w = 2 * h
  rows = _ROWS
  n_tiles = n // rows
  assert n % rows == 0

  nw = _NC * _NS
  chunk = 80
  sbs = 25
  nsb = e // nw // chunk // sbs
  src = edge_index[0].reshape(nw, nsb, sbs, chunk)
  dst = edge_index[1].reshape(nw, nsb, sbs, chunk)
  zeros = jnp.zeros((n, h), jnp.float32)
  batch3 = batch.reshape(n_tiles, 1, rows)
  wf1a = Wf1[:h]
  wf1b = Wf1[h:]

  seg_sum = _make_seg_sum(n, e, h)

  row_spec = pl.BlockSpec((rows, h), lambda i: (i, 0))
  agg_spec = pl.BlockSpec((_NC, rows, h), lambda i: (0, i, 0))

  h0 = pl.pallas_call(
      _embed_body,
      grid=(n_tiles,),
      in_specs=[pl.BlockSpec((rows, d), lambda i: (i, 0)),
                _full((d, h)), _full((h,))],
      out_specs=row_spec,
      out_shape=jax.ShapeDtypeStruct((n, h), jnp.float32),
  )(x, W_embed, b_embed)

  agg0 = seg_sum(h0, src, dst, zeros)

  h1 = pl.pallas_call(
      _gin_body,
      grid=(n_tiles,),
      in_specs=[agg_spec, _full((h, h2w)), _full((h2w,)),
                _full((h2w, h)), _full((h,))],
      out_specs=row_spec,
      out_shape=jax.ShapeDtypeStruct((n, h), jnp.float32),
  )(agg0, W1_0, b1_0, W2_0, b2_0)

  agg1 = seg_sum(h1, src, dst, zeros)

  z, sums = pl.pallas_call(
      functools.partial(_gin_fuse_body, n_tiles, g),
      grid=(n_tiles,),
      in_specs=[agg_spec, _full((h, h2w)), _full((h2w,)),
                _full((h2w, h)), _full((h,)),
                pl.BlockSpec((1, 1, rows), lambda i: (i, 0, 0)),
                _full((g, out_dim)), _full((h, h2w)),
                _full((out_dim, h2w)), _full((h2w,))],
      out_specs=[pl.BlockSpec((rows, h2w), lambda i: (i, 0)),
                 _full((2, h2w))],
      out_shape=[jax.ShapeDtypeStruct((n, h2w), jnp.float32),
                 jax.ShapeDtypeStruct((2, h2w), jnp.float32)],
  )(agg1, W1_1, b1_1, W2_1, b2_1, batch3, y_pred, wf1a, wf1b, bf1)

  node_score = pl.pallas_call(
      functools.partial(_fin_body, float(n)),
      grid=(n_tiles,),
      in_specs=[pl.BlockSpec((rows, h2w), lambda i: (i, 0)),
                _full((2, h2w)), _full((h2w,)), _full((h2w,)),
                _full((h2w, h)), _full((h,))],
      out_specs=row_spec,
      out_shape=jax.ShapeDtypeStruct((n, h), jnp.float32),
  )(z, sums, gamma, beta, Wf2, bf2)

  return node_score


# speedup vs baseline: 10.3645x; 1.1117x over previous
import functools

import jax
import jax.numpy as jnp
from jax import lax
from jax.experimental import pallas as pl
from jax.experimental.pallas import tpu as pltpu
from jax.experimental.pallas import tpu_sc as plsc

_NC = 2
_NS = 16
_ROWS = 400


def _make_seg_sum(n, e, h):
  nw = _NC * _NS
  epw = e // nw
  chunk = 80
  steps = epw // chunk
  sbs = 25
  assert epw % chunk == 0 and steps % sbs == 0
  spl = -(-n // _NS) // 8 * 8 + 8
  last = n - (_NS - 1) * spl
  assert last > 0 and last % 8 == 0

  def body(h_hbm, src_hbm, dst_hbm, zeros_hbm, out_hbm,
           acc, src_v, dst_v, rows_0, rows_1, rows_2,
           sem_0, sem_1, sem_2):
    cid = lax.axis_index("c")
    sid = lax.axis_index("s")
    row0 = pl.multiple_of(sid * spl, 8)

    def seed(rows):
      @pl.when(cid == 0)
      def _():
        pltpu.sync_copy(h_hbm.at[pl.ds(row0, rows)],
                        acc.at[pl.ds(row0, rows)])

      @pl.when(cid != 0)
      def _():
        pltpu.sync_copy(zeros_hbm.at[pl.ds(row0, rows)],
                        acc.at[pl.ds(row0, rows)])

    @pl.when(sid < _NS - 1)
    def _():
      seed(spl)

    @pl.when(sid == _NS - 1)
    def _():
      seed(last)

    plsc.subcore_barrier()

    wid = sid * _NC + cid
    bufs = (rows_0, rows_1, rows_2)
    sems = (sem_0, sem_1, sem_2)

    def start_gather(i, k):
      pltpu.async_copy(h_hbm.at[src_v.at[i]], bufs[k], sems[k])

    def wait_gather(i, k):
      pltpu.make_async_copy(h_hbm.at[src_v.at[i]], bufs[k], sems[k]).wait()

    def start_scatter(i, k):
      pltpu.async_copy(bufs[k], acc.at[dst_v.at[i]], sems[k], add=True)

    def wait_scatter(i, k):
      pltpu.make_async_copy(bufs[k], acc.at[dst_v.at[i]], sems[k]).wait()

    def superblock(sb, carry):
      pltpu.sync_copy(src_hbm.at[wid, sb], src_v)
      pltpu.sync_copy(dst_hbm.at[wid, sb], dst_v)
      start_gather(0, 0)
      start_gather(1, 1)

      def step(i, c2):
        m = lax.rem(i, 3)
        for k in range(3):
          @pl.when(m == k)
          def _(k=k):
            wait_gather(i, k)
            start_scatter(i, k)

            @pl.when(i + 2 < sbs)
            def _():
              @pl.when(i >= 1)
              def _():
                wait_scatter(i - 1, (k + 2) % 3)

              start_gather(i + 2, (k + 2) % 3)

        return c2

      lax.fori_loop(0, sbs, step, 0)
      for i in (sbs - 3, sbs - 2, sbs - 1):
        wait_scatter(i, i % 3)
      return carry

    lax.fori_loop(0, steps // sbs, superblock, 0)
    plsc.subcore_barrier()

    def writeout(rows):
      pltpu.sync_copy(acc.at[pl.ds(row0, rows)],
                      out_hbm.at[cid, pl.ds(row0, rows)])

    @pl.when(sid < _NS - 1)
    def _():
      writeout(spl)

    @pl.when(sid == _NS - 1)
    def _():
      writeout(last)

  return pl.kernel(
      body,
      out_type=jax.ShapeDtypeStruct((_NC, n, h), jnp.float32),
      mesh=plsc.VectorSubcoreMesh(core_axis_name="c", subcore_axis_name="s",
                                  num_cores=_NC, num_subcores=_NS),
      scratch_types=[
          pltpu.VMEM_SHARED((n, h), jnp.float32),
          pltpu.VMEM((sbs, chunk), jnp.int32),
          pltpu.VMEM((sbs, chunk), jnp.int32),
          pltpu.VMEM((chunk, h), jnp.float32),
          pltpu.VMEM((chunk, h), jnp.float32),
          pltpu.VMEM((chunk, h), jnp.float32),
          pltpu.SemaphoreType.DMA,
          pltpu.SemaphoreType.DMA,
          pltpu.SemaphoreType.DMA,
      ],
  )


def _dotf(a, b):
  return jnp.dot(a, b, preferred_element_type=jnp.float32)


def _embed_body(x_ref, w_ref, b_ref, o_ref):
  o_ref[...] = _dotf(x_ref[...], w_ref[...]) + b_ref[...]


def _gin_body(agg_ref, w1_ref, b1_ref, w2_ref, b2_ref, o_ref):
  t = agg_ref[0] + agg_ref[1]
  t = jnp.maximum(_dotf(t, w1_ref[...]) + b1_ref[...], 0.0)
  o_ref[...] = _dotf(t, w2_ref[...]) + b2_ref[...]


def _gin_fuse_body(n_tiles, g, agg_ref, w1_ref, b1_ref, w2_ref, b2_ref,
                   batch_ref, ypred_ref, wf1a_ref, wf1b_ref, bf1_ref,
                   z_ref, sums_ref):
  i = pl.program_id(0)
  t = agg_ref[0] + agg_ref[1]
  t = jnp.maximum(_dotf(t, w1_ref[...]) + b1_ref[...], 0.0)
  h2 = _dotf(t, w2_ref[...]) + b2_ref[...]
  yproj = _dotf(ypred_ref[...], wf1b_ref[...])
  b = batch_ref[0, 0, :]
  onehot = (b[:, None] ==
            lax.broadcasted_iota(jnp.int32, (b.shape[0], g), 1)
            ).astype(jnp.float32)
  z = _dotf(h2, wf1a_ref[...]) + _dotf(onehot, yproj) + bf1_ref[...]
  z_ref[...] = z

  @pl.when(i == 0)
  def _():
    sums_ref[...] = jnp.zeros_like(sums_ref)

  s1 = jnp.sum(z, axis=0)
  s2 = jnp.sum(z * z, axis=0)
  sums_ref[...] = sums_ref[...] + jnp.concatenate(
      [s1[None, :], s2[None, :]], axis=0)


def _fin_body(n, z_ref, sums_ref, gamma_ref, beta_ref, wf2_ref, bf2_ref,
              o_ref):
  mean = sums_ref[0, :] / n
  var = sums_ref[1, :] / n - mean * mean
  scale = lax.rsqrt(var + 1e-5) * gamma_ref[...]
  zn = (z_ref[...] - mean) * scale + beta_ref[...]
  zn = jnp.maximum(zn, 0.0)
  o = _dotf(zn, wf2_ref[...]) + bf2_ref[...]
  o_ref[...] = jax.nn.sigmoid(o)


def _full(shape):
  nd = len(shape)
  return pl.BlockSpec(shape, lambda i: (0,) * nd)


def kernel(x, edge_index, batch, y_pred, W_embed, b_embed,
           W1_0, b1_0, W2_0, b2_0, W1_1, b1_1, W2_1, b2_1,
           Wf1, bf1, gamma, beta, Wf2, bf2):
  n, d = x.shape
  h = W_embed.shape[1]
  e = edge_index.shape[1]
  g, out_dim = y_pred.shape
  h2w = 2 * h
  rows = _ROWS
  n_tiles = n // rows
  assert n % rows == 0

  nw = _NC * _NS
  chunk = 80
  sbs = 25
  nsb = e // nw // chunk // sbs
  src = edge_index[0].reshape(nw, nsb, sbs, chunk)
  dst = edge_index[1].reshape(nw, nsb, sbs, chunk)
  zeros = jnp.zeros((n, h), jnp.float32)
  batch3 = batch.reshape(n_tiles, 1, rows)
  wf1a = Wf1[:h]
  wf1b = Wf1[h:]

  seg_sum = _make_seg_sum(n, e, h)

  row_spec = pl.BlockSpec((rows, h), lambda i: (i, 0))
  agg_spec = pl.BlockSpec((_NC, rows, h), lambda i: (0, i, 0))

  h0 = pl.pallas_call(
      _embed_body,
      grid=(n_tiles,),
      in_specs=[pl.BlockSpec((rows, d), lambda i: (i, 0)),
                _full((d, h)), _full((h,))],
      out_specs=row_spec,
      out_shape=jax.ShapeDtypeStruct((n, h), jnp.float32),
  )(x, W_embed, b_embed)

  agg0 = seg_sum(h0, src, dst, zeros)

  h1 = pl.pallas_call(
      _gin_body,
      grid=(n_tiles,),
      in_specs=[agg_spec, _full((h, h2w)), _full((h2w,)),
                _full((h2w, h)), _full((h,))],
      out_specs=row_spec,
      out_shape=jax.ShapeDtypeStruct((n, h), jnp.float32),
  )(agg0, W1_0, b1_0, W2_0, b2_0)

  agg1 = seg_sum(h1, src, dst, zeros)

  z, sums = pl.pallas_call(
      functools.partial(_gin_fuse_body, n_tiles, g),
      grid=(n_tiles,),
      in_specs=[agg_spec, _full((h, h2w)), _full((h2w,)),
                _full((h2w, h)), _full((h,)),
                pl.BlockSpec((1, 1, rows), lambda i: (i, 0, 0)),
                _full((g, out_dim)), _full((h, h2w)),
                _full((out_dim, h2w)), _full((h2w,))],
      out_specs=[pl.BlockSpec((rows, h2w), lambda i: (i, 0)),
                 _full((2, h2w))],
      out_shape=[jax.ShapeDtypeStruct((n, h2w), jnp.float32),
                 jax.ShapeDtypeStruct((2, h2w), jnp.float32)],
  )(agg1, W1_1, b1_1, W2_1, b2_1, batch3, y_pred, wf1a, wf1b, bf1)

  node_score = pl.pallas_call(
      functools.partial(_fin_body, float(n)),
      grid=(n_tiles,),
      in_specs=[pl.BlockSpec((rows, h2w), lambda i: (i, 0)),
                _full((2, h2w)), _full((h2w,)), _full((h2w,)),
                _full((h2w, h)), _full((h,))],
      out_specs=row_spec,
      out_shape=jax.ShapeDtypeStruct((n, h), jnp.float32),
  )(z, sums, gamma, beta, Wf2, bf2)

  return node_score
